# baseline (speedup 1.0000x reference)
import jax
import jax.numpy as jnp
from jax import lax
from jax.experimental import pallas as pl
from jax.experimental.pallas import tpu as pltpu
from jax.experimental.pallas import tpu_sc as plsc

_B = 16384
_L = 100
_D = 64
_ROW = _L * _D
_NC = 2
_NS = 16
_NW = _NC * _NS
_RPW = _B // _NW
_G = 8
_LANES = 16

_DNUMS = lax.GatherDimensionNumbers(
    offset_dims=(), collapsed_slice_dims=(0,), start_index_map=(0,))


def _bcast_lane(vec, lane):
    idx = jnp.full((_LANES, 1), lane, dtype=jnp.int32)
    return lax.gather(vec, idx, _DNUMS, slice_sizes=(1,),
                      mode=lax.GatherScatterMode.PROMISE_IN_BOUNDS)


def _sc_body(x_hbm, w_hbm, out_hbm, xbuf, wbuf, obuf, sem0, sem1):
    wid = lax.axis_index("s") * _NC + lax.axis_index("c")
    base = wid * _RPW
    pltpu.sync_copy(w_hbm, wbuf)
    sems = (sem0, sem1)

    def compute_chunk(it, p):
        row0 = base + it * _G
        pltpu.sync_copy(x_hbm.at[pl.ds(row0, _G)], xbuf.at[p])

        @plsc.parallel_loop(0, _G, 1, unroll=2)
        def row_body(r):
            ro = p * (_G * _ROW) + r * _ROW
            for l0 in (0, 16, 32, 48, 64, 80, 84):
                lo = l0 if l0 != 84 else 96
                xv = xbuf[p, r, pl.ds(l0, _LANES)]
                xv = jnp.where(jnp.isnan(xv), 0.0, xv)
                for l in range(lo, min(l0 + _LANES, _L)):
                    xs = _bcast_lane(xv, l - l0)
                    for d in range(_D // _LANES):
                        off = l * _D + d * _LANES
                        wv = wbuf[pl.ds(off, _LANES)]
                        obuf[pl.ds(ro + off, _LANES)] = xs * wv

        pltpu.async_copy(obuf.at[pl.ds(p * (_G * _ROW), _G * _ROW)],
                         out_hbm.at[pl.ds(row0 * _ROW, _G * _ROW)], sems[p])

    def drain(p):
        pltpu.make_async_copy(
            obuf.at[pl.ds(p * (_G * _ROW), _G * _ROW)],
            out_hbm.at[pl.ds(base * _ROW, _G * _ROW)], sems[p]).wait()

    def loop_body(i, _):
        for p in (0, 1):
            @pl.when(i > 0)
            def _wait():
                drain(p)

            compute_chunk(2 * i + p, p)
        return _

    lax.fori_loop(0, _RPW // _G // 2, loop_body, 0)
    for p in (0, 1):
        drain(p)


def kernel(x, emb_weight):
    B, L = x.shape
    D = emb_weight.shape[1]
    wflat = emb_weight[1:L + 1].reshape(L * D)
    mesh = plsc.VectorSubcoreMesh(core_axis_name="c", subcore_axis_name="s")
    run = pl.kernel(
        _sc_body,
        mesh=mesh,
        out_type=jax.ShapeDtypeStruct((B * L * D,), x.dtype),
        scratch_types=[
            pltpu.VMEM((2, _G, L), jnp.float32),
            pltpu.VMEM((L * D,), jnp.float32),
            pltpu.VMEM((2 * _G * _ROW,), jnp.float32),
            pltpu.SemaphoreType.DMA,
            pltpu.SemaphoreType.DMA,
        ],
    )
    return run(x, wflat).reshape(B, L, D)


# speedup vs baseline: 1.8727x; 1.8727x over previous
import jax
import jax.numpy as jnp
from jax import lax
from jax.experimental import pallas as pl
from jax.experimental.pallas import tpu as pltpu
from jax.experimental.pallas import tpu_sc as plsc

_B = 16384
_L = 100
_D = 64
_NC = 2
_NS = 16
_NW = _NC * _NS
_RPW = _B // _NW
_G = 4
_LANES = 16

_DNUMS = lax.GatherDimensionNumbers(
    offset_dims=(), collapsed_slice_dims=(0,), start_index_map=(0,))


def _bcast_lane(vec, lane):
    idx = jnp.full((_LANES, 1), lane, dtype=jnp.int32)
    return lax.gather(vec, idx, _DNUMS, slice_sizes=(1,),
                      mode=lax.GatherScatterMode.PROMISE_IN_BOUNDS)


def _sc_body(x_hbm, w_hbm, out_hbm, xbuf, wbuf, obuf, sem0, sem1):
    wid = lax.axis_index("s") * _NC + lax.axis_index("c")
    base = wid * _RPW
    pltpu.sync_copy(w_hbm, wbuf)
    sems = (sem0, sem1)

    def compute_chunk(it, p):
        row0 = base + it * _G
        pltpu.sync_copy(x_hbm.at[pl.ds(row0, _G)], xbuf.at[p])

        for l0 in (0, 16, 32, 48, 64, 80, 84):
            lo = l0 if l0 != 84 else 96

            @plsc.parallel_loop(0, _G, 1, unroll=2)
            def row_body(r):
                xv = xbuf[p, r, pl.ds(l0, _LANES)]
                xv = jnp.where(jnp.isnan(xv), 0.0, xv)
                for l in range(lo, min(l0 + _LANES, _L)):
                    xs = _bcast_lane(xv, l - l0)
                    for d in range(_D // _LANES):
                        wv = wbuf[pl.ds(l * _D + d * _LANES, _LANES)]
                        obuf[p, r, l, pl.ds(d * _LANES, _LANES)] = xs * wv

        pltpu.async_copy(obuf.at[p], out_hbm.at[pl.ds(row0, _G)], sems[p])

    def drain(p):
        pltpu.make_async_copy(
            obuf.at[p], out_hbm.at[pl.ds(base, _G)], sems[p]).wait()

    def loop_body(i, _):
        for p in (0, 1):
            @pl.when(i > 0)
            def _wait():
                drain(p)

            compute_chunk(2 * i + p, p)
        return _

    lax.fori_loop(0, _RPW // _G // 2, loop_body, 0)
    for p in (0, 1):
        drain(p)


def kernel(x, emb_weight):
    B, L = x.shape
    D = emb_weight.shape[1]
    wflat = emb_weight[1:L + 1].reshape(L * D)
    mesh = plsc.VectorSubcoreMesh(core_axis_name="c", subcore_axis_name="s")
    run = pl.kernel(
        _sc_body,
        mesh=mesh,
        out_type=jax.ShapeDtypeStruct((B, L, D), x.dtype),
        scratch_types=[
            pltpu.VMEM((2, _G, L), jnp.float32),
            pltpu.VMEM((L * D,), jnp.float32),
            pltpu.VMEM((2, _G, L, D), jnp.float32),
            pltpu.SemaphoreType.DMA,
            pltpu.SemaphoreType.DMA,
        ],
    )
    return run(x, wflat)


# speedup vs baseline: 2.3240x; 1.2410x over previous
import jax
import jax.numpy as jnp
from jax import lax
from jax.experimental import pallas as pl
from jax.experimental.pallas import tpu as pltpu
from jax.experimental.pallas import tpu_sc as plsc

_B = 16384
_L = 100
_D = 64
_NC = 2
_NS = 16
_NW = _NC * _NS
_RPW = _B // _NW
_G = 4
_LANES = 16

_DNUMS = lax.GatherDimensionNumbers(
    offset_dims=(), collapsed_slice_dims=(0,), start_index_map=(0,))


def _bcast_lane(vec, lane):
    idx = jnp.full((_LANES, 1), lane, dtype=jnp.int32)
    return lax.gather(vec, idx, _DNUMS, slice_sizes=(1,),
                      mode=lax.GatherScatterMode.PROMISE_IN_BOUNDS)


def _sc_body(x_hbm, w_hbm, out_hbm, xbuf, wbuf, obuf, sem0, sem1):
    wid = lax.axis_index("s") * _NC + lax.axis_index("c")
    base = wid * _RPW
    pltpu.sync_copy(w_hbm, wbuf)
    sems = (sem0, sem1)

    def compute_chunk(it, p):
        row0 = base + it * _G
        pltpu.sync_copy(x_hbm.at[pl.ds(row0, _G)], xbuf.at[p])

        for l0 in (0, 16, 32, 48, 64, 80, 84):
            lo = l0 if l0 != 84 else 96

            @plsc.parallel_loop(0, _G, 1, unroll=4)
            def row_body(r):
                xv = xbuf[p, r, pl.ds(l0, _LANES)]
                xv = jnp.where(jnp.isnan(xv), 0.0, xv)
                for l in range(lo, min(l0 + _LANES, _L)):
                    xs = _bcast_lane(xv, l - l0)
                    for d in range(_D // _LANES):
                        wv = wbuf[pl.ds(l * _D + d * _LANES, _LANES)]
                        obuf[p, r, l, pl.ds(d * _LANES, _LANES)] = xs * wv

        pltpu.async_copy(obuf.at[p], out_hbm.at[pl.ds(row0, _G)], sems[p])

    def drain(p):
        pltpu.make_async_copy(
            obuf.at[p], out_hbm.at[pl.ds(base, _G)], sems[p]).wait()

    def loop_body(i, _):
        for p in (0, 1):
            @pl.when(i > 0)
            def _wait():
                drain(p)

            compute_chunk(2 * i + p, p)
        return _

    lax.fori_loop(0, _RPW // _G // 2, loop_body, 0)
    for p in (0, 1):
        drain(p)


def kernel(x, emb_weight):
    B, L = x.shape
    D = emb_weight.shape[1]
    wflat = emb_weight[1:L + 1].reshape(L * D)
    mesh = plsc.VectorSubcoreMesh(core_axis_name="c", subcore_axis_name="s")
    run = pl.kernel(
        _sc_body,
        mesh=mesh,
        out_type=jax.ShapeDtypeStruct((B, L, D), x.dtype),
        scratch_types=[
            pltpu.VMEM((2, _G, L), jnp.float32),
            pltpu.VMEM((L * D,), jnp.float32),
            pltpu.VMEM((2, _G, L, D), jnp.float32),
            pltpu.SemaphoreType.DMA,
            pltpu.SemaphoreType.DMA,
        ],
    )
    return run(x, wflat)


# speedup vs baseline: 2.6921x; 1.1584x over previous
import jax
import jax.numpy as jnp
from jax import lax
from jax.experimental import pallas as pl
from jax.experimental.pallas import tpu as pltpu
from jax.experimental.pallas import tpu_sc as plsc

_B = 16384
_L = 100
_D = 64
_H = _L // 2
_W = 2 * _D
_NC = 2
_NS = 16
_NW = _NC * _NS
_RPW = _B // _NW
_G = 8
_LANES = 16

_DNUMS = lax.GatherDimensionNumbers(
    offset_dims=(), collapsed_slice_dims=(0,), start_index_map=(0,))


def _bcast_lane(vec, lane):
    idx = jnp.full((_LANES, 1), lane, dtype=jnp.int32)
    return lax.gather(vec, idx, _DNUMS, slice_sizes=(1,),
                      mode=lax.GatherScatterMode.PROMISE_IN_BOUNDS)


def _sc_body(x_hbm, w_hbm, out_hbm, xbuf, wbuf, obuf, sem0, sem1):
    wid = lax.axis_index("s") * _NC + lax.axis_index("c")
    base = wid * _RPW
    pltpu.sync_copy(w_hbm, wbuf)
    sems = (sem0, sem1)

    def compute_chunk(it, p):
        row0 = base + it * _G
        pltpu.sync_copy(x_hbm.at[pl.ds(row0, _G)], xbuf.at[p])

        for l0 in (0, 16, 32, 48, 64, 80, 84):
            j0 = 0 if l0 != 84 else 12

            @plsc.parallel_loop(0, _G, 1, unroll=4)
            def row_body(r):
                xv = xbuf[p, r, pl.ds(l0, _LANES)]
                xv = jnp.where(jnp.isnan(xv), 0.0, xv)
                for j in range(j0, _LANES, 2):
                    k = (l0 + j) // 2
                    xs0 = _bcast_lane(xv, j)
                    xs1 = _bcast_lane(xv, j + 1)
                    for g in range(_W // _LANES):
                        off = k * _W + g * _LANES
                        wv = wbuf[pl.ds(off, _LANES)]
                        xs = xs0 if g < 4 else xs1
                        obuf[p, r, k, pl.ds(g * _LANES, _LANES)] = xs * wv

        pltpu.async_copy(obuf.at[p], out_hbm.at[pl.ds(row0, _G)], sems[p])

    def drain(p):
        pltpu.make_async_copy(
            obuf.at[p], out_hbm.at[pl.ds(base, _G)], sems[p]).wait()

    def loop_body(i, _):
        for p in (0, 1):
            @pl.when(i > 0)
            def _wait():
                drain(p)

            compute_chunk(2 * i + p, p)
        return _

    lax.fori_loop(0, _RPW // _G // 2, loop_body, 0)
    for p in (0, 1):
        drain(p)


def kernel(x, emb_weight):
    B, L = x.shape
    D = emb_weight.shape[1]
    wflat = emb_weight[1:L + 1].reshape(L * D)
    mesh = plsc.VectorSubcoreMesh(core_axis_name="c", subcore_axis_name="s")
    run = pl.kernel(
        _sc_body,
        mesh=mesh,
        out_type=jax.ShapeDtypeStruct((B, _H, _W), x.dtype),
        scratch_types=[
            pltpu.VMEM((2, _G, L), jnp.float32),
            pltpu.VMEM((L * D,), jnp.float32),
            pltpu.VMEM((2, _G, _H, _W), jnp.float32),
            pltpu.SemaphoreType.DMA,
            pltpu.SemaphoreType.DMA,
        ],
    )
    return run(x, wflat).reshape(B, L, D)


# speedup vs baseline: 3.5259x; 1.3097x over previous
import jax
import jax.numpy as jnp
from jax import lax
from jax.experimental import pallas as pl
from jax.experimental.pallas import tpu as pltpu
from jax.experimental.pallas import tpu_sc as plsc

_B = 16384
_L = 100
_D = 64
_H = _L // 2
_W = 2 * _D
_NC = 2
_NS = 16
_NW = _NC * _NS
_RPW = _B // _NW
_G = 8
_LANES = 16

_DNUMS = lax.GatherDimensionNumbers(
    offset_dims=(), collapsed_slice_dims=(0,), start_index_map=(0,))


def _bcast_lane(vec, lane):
    idx = jnp.full((_LANES, 1), lane, dtype=jnp.int32)
    return lax.gather(vec, idx, _DNUMS, slice_sizes=(1,),
                      mode=lax.GatherScatterMode.PROMISE_IN_BOUNDS)


def _sc_body(x_hbm, w_hbm, out_hbm, xbuf, wbuf, obuf, sem0, sem1):
    wid = lax.axis_index("s") * _NC + lax.axis_index("c")
    base = wid * _RPW
    pltpu.sync_copy(w_hbm, wbuf)
    sems = (sem0, sem1)

    def compute_chunk(it, p):
        row0 = base + it * _G
        pltpu.sync_copy(x_hbm.at[pl.ds(row0, _G)], xbuf.at[p])

        for l0 in (0, 16, 32, 48, 64, 80, 84):
            k0, nk = (l0 // 2, 8) if l0 != 84 else (48, 2)
            xvs = []
            for r in range(_G):
                xv = xbuf[p, r, pl.ds(l0, _LANES)]
                xvs.append(jnp.where(jnp.isnan(xv), 0.0, xv))

            @plsc.parallel_loop(0, nk, 1, unroll=2)
            def k_body(kk):
                k = k0 + kk
                j = 2 * kk + (0 if l0 != 84 else 12)
                wvs = [wbuf[pl.ds(k * _W + g * _LANES, _LANES)]
                       for g in range(_W // _LANES)]
                xs0s = [_bcast_lane(xvs[r], j) for r in range(_G)]
                xs1s = [_bcast_lane(xvs[r], j + 1) for r in range(_G)]
                for r in range(_G):
                    for g in range(_W // _LANES):
                        xs = xs0s[r] if g < 4 else xs1s[r]
                        obuf[p, r, k, pl.ds(g * _LANES, _LANES)] = xs * wvs[g]

        pltpu.async_copy(obuf.at[p], out_hbm.at[pl.ds(row0, _G)], sems[p])

    def drain(p):
        pltpu.make_async_copy(
            obuf.at[p], out_hbm.at[pl.ds(base, _G)], sems[p]).wait()

    def loop_body(i, _):
        for p in (0, 1):
            @pl.when(i > 0)
            def _wait():
                drain(p)

            compute_chunk(2 * i + p, p)
        return _

    lax.fori_loop(0, _RPW // _G // 2, loop_body, 0)
    for p in (0, 1):
        drain(p)


def kernel(x, emb_weight):
    B, L = x.shape
    D = emb_weight.shape[1]
    wflat = emb_weight[1:L + 1].reshape(L * D)
    mesh = plsc.VectorSubcoreMesh(core_axis_name="c", subcore_axis_name="s")
    run = pl.kernel(
        _sc_body,
        mesh=mesh,
        out_type=jax.ShapeDtypeStruct((B, _H, _W), x.dtype),
        scratch_types=[
            pltpu.VMEM((2, _G, L), jnp.float32),
            pltpu.VMEM((L * D,), jnp.float32),
            pltpu.VMEM((2, _G, _H, _W), jnp.float32),
            pltpu.SemaphoreType.DMA,
            pltpu.SemaphoreType.DMA,
        ],
    )
    return run(x, wflat).reshape(B, L, D)


# speedup vs baseline: 3.6563x; 1.0370x over previous
import jax
import jax.numpy as jnp
from jax import lax
from jax.experimental import pallas as pl
from jax.experimental.pallas import tpu as pltpu
from jax.experimental.pallas import tpu_sc as plsc

_B = 16384
_L = 100
_D = 64
_H = _L // 2
_W = 2 * _D
_NC = 2
_NS = 16
_NW = _NC * _NS
_RPW = _B // _NW
_G = 8
_LANES = 16

_DNUMS = lax.GatherDimensionNumbers(
    offset_dims=(), collapsed_slice_dims=(0,), start_index_map=(0,))


def _bcast_lane(vec, lane):
    idx = jnp.full((_LANES, 1), lane, dtype=jnp.int32)
    return lax.gather(vec, idx, _DNUMS, slice_sizes=(1,),
                      mode=lax.GatherScatterMode.PROMISE_IN_BOUNDS)


def _sc_body(x_hbm, w_hbm, out_hbm, xbuf, wbuf, obuf, sem0, sem1):
    wid = lax.axis_index("s") * _NC + lax.axis_index("c")
    base = wid * _RPW
    pltpu.sync_copy(w_hbm, wbuf)
    sems = (sem0, sem1)

    def compute_chunk(it, p):
        row0 = base + it * _G
        pltpu.sync_copy(x_hbm.at[pl.ds(row0, _G)], xbuf.at[p])

        for l0 in (0, 16, 32, 48, 64, 80, 84):
            k0, nk = (l0 // 2, 8) if l0 != 84 else (48, 2)
            xvs = []
            for r in range(_G):
                xv = xbuf[p, r, pl.ds(l0, _LANES)]
                xvs.append(jnp.where(jnp.isnan(xv), 0.0, xv))

            @plsc.parallel_loop(0, nk, 1, unroll=4)
            def k_body(kk):
                k = k0 + kk
                j = 2 * kk + (0 if l0 != 84 else 12)
                wvs = [wbuf[pl.ds(k * _W + g * _LANES, _LANES)]
                       for g in range(_W // _LANES)]
                xs0s = [_bcast_lane(xvs[r], j) for r in range(_G)]
                xs1s = [_bcast_lane(xvs[r], j + 1) for r in range(_G)]
                for r in range(_G):
                    for g in range(_W // _LANES):
                        xs = xs0s[r] if g < 4 else xs1s[r]
                        obuf[p, r, k, pl.ds(g * _LANES, _LANES)] = xs * wvs[g]

        pltpu.async_copy(obuf.at[p], out_hbm.at[pl.ds(row0, _G)], sems[p])

    def drain(p):
        pltpu.make_async_copy(
            obuf.at[p], out_hbm.at[pl.ds(base, _G)], sems[p]).wait()

    def loop_body(i, _):
        for p in (0, 1):
            @pl.when(i > 0)
            def _wait():
                drain(p)

            compute_chunk(2 * i + p, p)
        return _

    lax.fori_loop(0, _RPW // _G // 2, loop_body, 0)
    for p in (0, 1):
        drain(p)


def kernel(x, emb_weight):
    B, L = x.shape
    D = emb_weight.shape[1]
    wflat = emb_weight[1:L + 1].reshape(L * D)
    mesh = plsc.VectorSubcoreMesh(core_axis_name="c", subcore_axis_name="s")
    run = pl.kernel(
        _sc_body,
        mesh=mesh,
        out_type=jax.ShapeDtypeStruct((B, _H, _W), x.dtype),
        scratch_types=[
            pltpu.VMEM((2, _G, L), jnp.float32),
            pltpu.VMEM((L * D,), jnp.float32),
            pltpu.VMEM((2, _G, _H, _W), jnp.float32),
            pltpu.SemaphoreType.DMA,
            pltpu.SemaphoreType.DMA,
        ],
    )
    return run(x, wflat).reshape(B, L, D)
